# baseline (speedup 1.0000x reference)
import functools

import jax
import jax.numpy as jnp
from jax import lax
from jax.experimental import pallas as pl
from jax.experimental.pallas import tpu as pltpu
from jax.experimental.pallas import tpu_sc as plsc

NUM_ROWS = 16384
NUM_CLASSES = 1000
LANES = 16
TILE_W = 128
GROUPS = TILE_W // LANES

_info = plsc.get_sparse_core_info()
NC, NS = _info.num_cores, _info.num_subcores
NW = NC * NS

SC_SAMPLES = 8192
TC_SAMPLES = NUM_ROWS - SC_SAMPLES
SC_BASE = TC_SAMPLES
SPW = SC_SAMPLES // NW
TCPW = SPW // TILE_W
CT_PER_CHUNK = 25
CLS_PER_CHUNK = CT_PER_CHUNK * 8
PARTS = NUM_CLASSES // CLS_PER_CHUNK
NCHUNK = TCPW * PARTS
assert NCHUNK % 2 == 0


def _make_sc_kernel():
    mesh = plsc.VectorSubcoreMesh(core_axis_name="c", subcore_axis_name="s")

    @functools.partial(
        pl.kernel,
        mesh=mesh,
        out_type=jax.ShapeDtypeStruct((SC_SAMPLES,), jnp.float32),
        compiler_params=pltpu.CompilerParams(
            needs_layout_passes=False, use_tc_tiling_on_sc=True,
            skip_device_barrier=True),
        scratch_types=[
            pltpu.VMEM((CLS_PER_CHUNK, TILE_W), jnp.float32),
            pltpu.VMEM((CLS_PER_CHUNK, TILE_W), jnp.float32),
            pltpu.VMEM((SPW,), jnp.float32),
            pltpu.SemaphoreType.DMA,
            pltpu.SemaphoreType.DMA,
        ],
    )
    def body(xt_hbm, out_hbm, buf_a, buf_b, out_v, sem_a, sem_b):
        wid = lax.axis_index("s") * NC + lax.axis_index("c")
        col_base = SC_BASE + wid * SPW

        zeros = jnp.zeros((LANES,), jnp.float32)
        for i in range(SPW // LANES):
            out_v[pl.ds(i * LANES, LANES)] = zeros

        def chunk_src(c, buf, sem):
            src = xt_hbm.at[
                pl.ds((c % PARTS) * CLS_PER_CHUNK, CLS_PER_CHUNK),
                pl.ds(col_base + (c // PARTS) * TILE_W, TILE_W),
            ]
            return pltpu.make_async_copy(src, buf, sem)

        def compute(buf, c):
            col0 = (c // PARTS) * TILE_W
            cls0 = (c % PARTS) * CLS_PER_CHUNK

            def t_body(t, accs):
                out = list(accs)
                for s in range(4):
                    w = (cls0 + 4 * t + s).astype(jnp.float32)
                    for g in range(GROUPS):
                        x = buf[4 * t + s, pl.ds(g * LANES, LANES)]
                        out[g] = out[g] + x * w
                return tuple(out)

            accs = tuple(
                out_v[pl.ds(col0 + g * LANES, LANES)] for g in range(GROUPS))
            accs = lax.fori_loop(0, CLS_PER_CHUNK // 4, t_body, accs)
            for g in range(GROUPS):
                out_v[pl.ds(col0 + g * LANES, LANES)] = accs[g]

        chunk_src(0, buf_a, sem_a).start()

        def loop_body(i, _):
            c0 = 2 * i
            c1 = 2 * i + 1
            chunk_src(c1, buf_b, sem_b).start()
            chunk_src(c0, buf_a, sem_a).wait()
            compute(buf_a, c0)
            chunk_src(jnp.minimum(c1 + 1, NCHUNK - 1), buf_a, sem_a).start()
            chunk_src(c1, buf_b, sem_b).wait()
            compute(buf_b, c1)
            return 0

        lax.fori_loop(0, NCHUNK // 2, loop_body, 0)
        chunk_src(NCHUNK - 1, buf_a, sem_a).wait()

        pltpu.sync_copy(out_v, out_hbm.at[pl.ds(wid * SPW, SPW)])

    return body


TC_SB = 2048
TC_GRID = (TC_SAMPLES // TC_SB,)


def _tc_body(x_ref, o_ref):
    w = lax.broadcasted_iota(
        jnp.int32, (NUM_CLASSES, 1), 0).astype(jnp.float32)
    o_ref[...] = jnp.sum(x_ref[...] * w, axis=0)


_tc_kernel = pl.pallas_call(
    _tc_body,
    grid=TC_GRID,
    in_specs=[pl.BlockSpec((NUM_CLASSES, TC_SB), lambda i: (0, i))],
    out_specs=pl.BlockSpec((TC_SB,), lambda i: (i,)),
    out_shape=jax.ShapeDtypeStruct((TC_SAMPLES,), jnp.float32),
    compiler_params=pltpu.CompilerParams(
        dimension_semantics=("parallel",), skip_device_barrier=True),
)

_sc_kernel = _make_sc_kernel()


def kernel(onehot):
    xt = onehot.T
    sc_out = _sc_kernel(xt)
    tc_out = _tc_kernel(xt)
    return jnp.concatenate([tc_out, sc_out]).astype(jnp.int32)


# speedup vs baseline: 1.1069x; 1.1069x over previous
import functools

import jax
import jax.numpy as jnp
from jax import lax
from jax.experimental import pallas as pl
from jax.experimental.pallas import tpu as pltpu
from jax.experimental.pallas import tpu_sc as plsc

NUM_ROWS = 16384
NUM_CLASSES = 1000
LANES = 16
TILE_W = 128
GROUPS = TILE_W // LANES

_info = plsc.get_sparse_core_info()
NC, NS = _info.num_cores, _info.num_subcores
NW = NC * NS

SC_SAMPLES = 4096
TC_SAMPLES = NUM_ROWS - SC_SAMPLES
SC_BASE = TC_SAMPLES
SPW = SC_SAMPLES // NW
TCPW = SPW // TILE_W
CT_PER_CHUNK = 25
CLS_PER_CHUNK = CT_PER_CHUNK * 8
PARTS = NUM_CLASSES // CLS_PER_CHUNK
NCHUNK = TCPW * PARTS
assert NCHUNK % 2 == 1


def _make_sc_kernel():
    mesh = plsc.VectorSubcoreMesh(core_axis_name="c", subcore_axis_name="s")

    @functools.partial(
        pl.kernel,
        mesh=mesh,
        out_type=jax.ShapeDtypeStruct((SC_SAMPLES,), jnp.float32),
        compiler_params=pltpu.CompilerParams(
            needs_layout_passes=False, use_tc_tiling_on_sc=True,
            skip_device_barrier=True),
        scratch_types=[
            pltpu.VMEM((CLS_PER_CHUNK, TILE_W), jnp.float32),
            pltpu.VMEM((CLS_PER_CHUNK, TILE_W), jnp.float32),
            pltpu.VMEM((SPW,), jnp.float32),
            pltpu.SemaphoreType.DMA,
            pltpu.SemaphoreType.DMA,
        ],
    )
    def body(xt_hbm, out_hbm, buf_a, buf_b, out_v, sem_a, sem_b):
        wid = lax.axis_index("s") * NC + lax.axis_index("c")
        col_base = SC_BASE + wid * SPW

        zeros = jnp.zeros((LANES,), jnp.float32)
        for i in range(SPW // LANES):
            out_v[pl.ds(i * LANES, LANES)] = zeros

        def chunk_src(c, buf, sem):
            src = xt_hbm.at[
                pl.ds((c % PARTS) * CLS_PER_CHUNK, CLS_PER_CHUNK),
                pl.ds(col_base + (c // PARTS) * TILE_W, TILE_W),
            ]
            return pltpu.make_async_copy(src, buf, sem)

        def compute(buf, c):
            col0 = (c // PARTS) * TILE_W
            cls0 = (c % PARTS) * CLS_PER_CHUNK

            def t_body(t, accs):
                out = list(accs)
                for s in range(2):
                    w = (cls0 + 2 * t + s).astype(jnp.float32)
                    for g in range(GROUPS):
                        x = buf[2 * t + s, pl.ds(g * LANES, LANES)]
                        out[g] = out[g] + x * w
                return tuple(out)

            accs = tuple(
                out_v[pl.ds(col0 + g * LANES, LANES)] for g in range(GROUPS))
            accs = lax.fori_loop(0, CLS_PER_CHUNK // 2, t_body, accs)
            for g in range(GROUPS):
                out_v[pl.ds(col0 + g * LANES, LANES)] = accs[g]

        chunk_src(0, buf_a, sem_a).start()

        def loop_body(i, _):
            c0 = 2 * i
            c1 = 2 * i + 1
            chunk_src(c1, buf_b, sem_b).start()
            chunk_src(c0, buf_a, sem_a).wait()
            compute(buf_a, c0)
            chunk_src(jnp.minimum(c1 + 1, NCHUNK - 1), buf_a, sem_a).start()
            chunk_src(c1, buf_b, sem_b).wait()
            compute(buf_b, c1)
            return 0

        lax.fori_loop(0, NCHUNK // 2, loop_body, 0)
        chunk_src(NCHUNK - 1, buf_a, sem_a).wait()
        compute(buf_a, NCHUNK - 1)

        pltpu.sync_copy(out_v, out_hbm.at[pl.ds(wid * SPW, SPW)])

    return body


TC_SB = 2048
TC_GRID = (TC_SAMPLES // TC_SB,)


def _tc_body(x_ref, o_ref):
    w = lax.broadcasted_iota(
        jnp.int32, (NUM_CLASSES, 1), 0).astype(jnp.float32)
    o_ref[...] = jnp.sum(x_ref[...] * w, axis=0)


_tc_kernel = pl.pallas_call(
    _tc_body,
    grid=TC_GRID,
    in_specs=[pl.BlockSpec((NUM_CLASSES, TC_SB), lambda i: (0, i))],
    out_specs=pl.BlockSpec((TC_SB,), lambda i: (i,)),
    out_shape=jax.ShapeDtypeStruct((TC_SAMPLES,), jnp.float32),
    compiler_params=pltpu.CompilerParams(
        dimension_semantics=("parallel",), skip_device_barrier=True),
)

_sc_kernel = _make_sc_kernel()


def kernel(onehot):
    xt = onehot.T
    sc_out = _sc_kernel(xt)
    tc_out = _tc_kernel(xt)
    return jnp.concatenate([tc_out, sc_out]).astype(jnp.int32)
